# baseline (speedup 1.0000x reference)
import functools

import jax
import jax.numpy as jnp
from jax import lax
from jax.experimental import pallas as pl
from jax.experimental.pallas import tpu as pltpu
from jax.experimental.pallas import tpu_sc as plsc

_NC = 2
_NS = 16
_NW = _NC * _NS

_V = 1000000
_CW = 256
_NCH_FULL = _V // _CW
_RAG = _V - _NCH_FULL * _CW
_CAP_I = 128
_CAP_U = 16
_BL = 204800
_B = 4096
_TRASH = _BL + _B
_OUT_ROWS = 211200


def _sc_gather(item_ids, user_ids, mfT_i, mlpT_i, mfT_u, mlpT_u,
               rag_mf_i, rag_mlp_i, rag_mf_u, rag_mlp_u):
  d = mfT_i.shape[0]
  i32 = jnp.int32

  mesh = plsc.VectorSubcoreMesh(core_axis_name="c", subcore_axis_name="s",
                                num_cores=_NC, num_subcores=_NS)

  @functools.partial(
      pl.kernel,
      out_type=jax.ShapeDtypeStruct((_OUT_ROWS, 2 * d), jnp.float32),
      mesh=mesh,
      compiler_params=pltpu.CompilerParams(use_tc_tiling_on_sc=True,
                                           needs_layout_passes=False),
      scratch_types=[
          pltpu.VMEM((4096,), i32),
          pltpu.VMEM((128, _CAP_I), i32),
          pltpu.VMEM((128, _CAP_U), i32),
          pltpu.VMEM((128,), i32),
          pltpu.VMEM((128,), i32),
          pltpu.VMEM((d, _CW), jnp.float32),
          pltpu.VMEM((d, _CW), jnp.float32),
          pltpu.VMEM((_CAP_I, 2 * d), jnp.float32),
          pltpu.VMEM((d, _RAG), jnp.float32),
          pltpu.VMEM((d, _RAG), jnp.float32),
          pltpu.VMEM((16, 8), i32),
          pltpu.SemaphoreType.DMA,
          pltpu.SemaphoreType.DMA,
      ],
  )
  def k(item_ids_h, user_ids_h, mfTi_h, mlpTi_h, mfTu_h, mlpTu_h,
        ragA_i_h, ragB_i_h, ragA_u_h, ragB_u_h, out_h,
        idbuf, match_i, match_u, tails_i, tails_u,
        slab_a, slab_b, rowbuf, rag_a, rag_b, posidx, semS, semW):
    wid = lax.axis_index("s") * _NC + lax.axis_index("c")
    nch = 122 + jnp.where(wid >= 30, 1, 0)
    start = 122 * wid + jnp.maximum(wid - 30, 0)
    lo = start * _CW
    hi = lo + nch * _CW + jnp.where(wid == 31, _RAG, 0)
    lanes = lax.broadcasted_iota(i32, (16,), 0)
    zeros16 = jnp.zeros((16,), i32)
    trash16 = jnp.full((16,), _TRASH, i32)

    for k16 in range(8):
      tails_i[pl.ds(16 * k16, 16)] = zeros16
      tails_u[pl.ds(16 * k16, 16)] = zeros16

    def scan_list(ids_h, nblk, pos_base, match, tails, cap):
      def blk(s, carry):
        pltpu.sync_copy(
            ids_h.at[pl.ds(pl.multiple_of(s * 4096, 4096), 4096)], idbuf)
        def step(t, c2):
          idv = idbuf[pl.ds(t * 16, 16)]
          posv = pos_base + s * 4096 + t * 16 + lanes
          m = (idv >= lo) & (idv < hi)
          local = idv - lo
          bkt = lax.shift_right_logical(local, 8)
          rank, lastm = plsc.scan_count(bkt, mask=m)
          tl = plsc.load_gather(tails, [bkt], mask=m)
          wr = tl + rank - 1
          mw = m & (wr < cap)
          packed = posv * 256 + (local & 255)
          plsc.store_scatter(match, [bkt, wr], packed, mask=mw)
          plsc.addupdate_scatter(tails, [bkt], rank, mask=m & lastm)
          return c2
        return lax.fori_loop(0, 256, step, carry)
      lax.fori_loop(0, nblk, blk, 0)

    scan_list(item_ids_h, _BL // 4096, 0, match_i, tails_i, _CAP_I)
    scan_list(user_ids_h, _B // 4096, _BL, match_u, tails_u, _CAP_U)

    def drain(n):
      for s in range(16):
        @pl.when(s < n)
        def _():
          pltpu.make_async_copy(rowbuf.at[pl.ds(0, 8)],
                                out_h.at[pl.ds(0, 8)], semW).wait()

    def process(cnt):
      for k16 in range(8):
        lv = k16 * 16 + lanes
        plsc.store_scatter(posidx, [lax.shift_right_logical(lv, 3), lv & 7],
                           trash16)
      jv_cnt = cnt

      def grp(g, c3, match, j, sa, sb):
        lv = g * 16 + lanes
        mk = lv < jv_cnt
        jb = jnp.full((16,), j, i32)
        packed = plsc.load_gather(match, [jb, lv], mask=mk)
        posv = lax.shift_right_logical(packed, 8)
        localv = packed & 255
        plsc.store_scatter(posidx, [lax.shift_right_logical(lv, 3), lv & 7],
                           posv, mask=mk)
        for r in range(0, d, 1):
          rb = jnp.full((16,), r, i32)
          va = plsc.load_gather(sa, [rb, localv], mask=mk)
          plsc.store_scatter(rowbuf, [lv, rb], va, mask=mk)
          vb = plsc.load_gather(sb, [rb, localv], mask=mk)
          plsc.store_scatter(rowbuf, [lv, rb + d], vb, mask=mk)
        return c3
      return grp

    def gather_pass(tA, tB, ragA, ragB, match, tails, cap):
      def chunk(j, nscat_prev):
        c0 = pl.multiple_of((start + j) * _CW, 128)
        cpa = pltpu.async_copy(tA.at[:, pl.ds(c0, _CW)], slab_a, semS)
        cpb = pltpu.async_copy(tB.at[:, pl.ds(c0, _CW)], slab_b, semS)
        drain(nscat_prev)
        cpa.wait()
        cpb.wait()
        cnt_vec = plsc.load_gather(tails, [jnp.full((16,), j, i32)])
        cnt = jnp.minimum(jnp.max(cnt_vec), cap)
        grp = process(cnt)
        lax.fori_loop(0, lax.div(cnt + 15, 16),
                      lambda g, c: grp(g, c, match, j, slab_a, slab_b), 0)
        nscat = lax.div(cnt + 7, 8)
        for s in range(16):
          @pl.when(s < nscat)
          def _():
            pltpu.async_copy(rowbuf.at[pl.ds(8 * s, 8)],
                             out_h.at[posidx.at[s]], semW)
        return nscat
      last = lax.fori_loop(0, nch, chunk, 0)
      drain(last)

      @pl.when(wid == 31)
      def _():
        cpa = pltpu.async_copy(ragA, rag_a, semS)
        cpb = pltpu.async_copy(ragB, rag_b, semS)
        cpa.wait()
        cpb.wait()
        cnt_vec = plsc.load_gather(tails, [jnp.full((16,), 123, i32)])
        cnt = jnp.minimum(jnp.max(cnt_vec), cap)
        grp = process(cnt)
        lax.fori_loop(0, lax.div(cnt + 15, 16),
                      lambda g, c: grp(g, c, match, 123, rag_a, rag_b), 0)
        nscat = lax.div(cnt + 7, 8)
        for s in range(16):
          @pl.when(s < nscat)
          def _():
            pltpu.async_copy(rowbuf.at[pl.ds(8 * s, 8)],
                             out_h.at[posidx.at[s]], semW)
        drain(nscat)

    gather_pass(mfTi_h, mlpTi_h, ragA_i_h, ragB_i_h, match_i, tails_i,
                _CAP_I)
    gather_pass(mfTu_h, mlpTu_h, ragA_u_h, ragB_u_h, match_u, tails_u,
                _CAP_U)

  return k(item_ids, user_ids, mfT_i, mlpT_i, mfT_u, mlpT_u,
           rag_mf_i, rag_mlp_i, rag_mf_u, rag_mlp_u)


def _tc_body(x_ref, xu_ref, w1u_ref, w1z_ref, b1_ref, w2_ref, b2_ref,
             wmfm_ref, wh2_ref, out_ref, *, bb, ll):
  f32 = jnp.float32
  dot = functools.partial(jnp.dot, preferred_element_type=f32)
  xu = xu_ref[...]
  a = dot(xu, w1u_ref[...]) + b1_ref[...]
  vmf = xu * wmfm_ref[...]
  r_i = lax.broadcasted_iota(jnp.int32, (bb * ll, bb), 0)
  b50 = lax.broadcasted_iota(jnp.int32, (bb * ll, bb), 1) * ll
  p = ((r_i >= b50) & (r_i < b50 + ll)).astype(f32)
  a_exp = dot(p, a)
  vmf_exp = dot(p, vmf)
  x = x_ref[...]
  h1 = jnp.maximum(dot(x, w1z_ref[...]) + a_exp, 0.0)
  h2 = jnp.maximum(dot(h1, w2_ref[...]) + b2_ref[...], 0.0)
  mf_c = jnp.sum(x * vmf_exp, axis=1, keepdims=True)
  out_ref[...] = mf_c + jnp.sum(h2 * wh2_ref[...], axis=1, keepdims=True)


def kernel(user, item, mf_user_em, mf_item_em, mlp_user_em, mlp_item_em,
           W1, b1, W2, b2, Wout):
  b, ll = item.shape
  d = mf_user_em.shape[1]
  bl = b * ll
  f32 = jnp.float32

  item_ids = item.reshape(-1)
  user_ids = user.reshape(-1)

  rag0 = _NCH_FULL * _CW
  out_all = _sc_gather(item_ids, user_ids, mf_item_em.T, mlp_item_em.T,
                       mf_user_em.T, mlp_user_em.T,
                       mf_item_em[rag0:, :].T, mlp_item_em[rag0:, :].T,
                       mf_user_em[rag0:, :].T, mlp_user_em[rag0:, :].T)

  zdd = jnp.zeros((d, d), f32)
  w1u = jnp.concatenate([zdd, W1[:d, :]], axis=0)
  w1z = jnp.concatenate([zdd, W1[d:, :]], axis=0)
  b1r = b1.reshape(1, d)
  b2r = b2.reshape(1, d // 2)
  wmfm = jnp.concatenate([Wout[:d, 0],
                          jnp.zeros((d,), f32)]).reshape(1, 2 * d)
  wh2 = Wout[d:, 0].reshape(1, d // 2)

  bb = 128
  n_blocks = b // bb
  rows = bb * ll

  grid_spec = pl.GridSpec(
      grid=(n_blocks,),
      in_specs=[
          pl.BlockSpec((rows, 2 * d), lambda i: (i, 0)),
          pl.BlockSpec((bb, 2 * d), lambda i: (_BL // bb + i, 0)),
          pl.BlockSpec((2 * d, d), lambda i: (0, 0)),
          pl.BlockSpec((2 * d, d), lambda i: (0, 0)),
          pl.BlockSpec((1, d), lambda i: (0, 0)),
          pl.BlockSpec((d, d // 2), lambda i: (0, 0)),
          pl.BlockSpec((1, d // 2), lambda i: (0, 0)),
          pl.BlockSpec((1, 2 * d), lambda i: (0, 0)),
          pl.BlockSpec((1, d // 2), lambda i: (0, 0)),
      ],
      out_specs=pl.BlockSpec((rows, 1), lambda i: (i, 0)),
  )

  out_flat = pl.pallas_call(
      functools.partial(_tc_body, bb=bb, ll=ll),
      grid_spec=grid_spec,
      out_shape=jax.ShapeDtypeStruct((bl, 1), jnp.float32),
      compiler_params=pltpu.CompilerParams(
          dimension_semantics=("parallel",)),
  )(out_all, out_all, w1u, w1z, b1r, W2, b2r, wmfm, wh2)

  return out_flat.reshape(b, ll, 1)


# speedup vs baseline: 1.1740x; 1.1740x over previous
import functools

import jax
import jax.numpy as jnp
from jax import lax
from jax.experimental import pallas as pl
from jax.experimental.pallas import tpu as pltpu
from jax.experimental.pallas import tpu_sc as plsc

_NC = 2
_NS = 16
_NW = _NC * _NS

_V = 1000000
_CW = 256
_NCH_FULL = _V // _CW
_RAG = _V - _NCH_FULL * _CW
_CAP_I = 128
_CAP_U = 16
_BL = 204800
_B = 4096
_TRASH = _BL + _B
_OUT_ROWS = 211200


def _sc_gather(item_ids, user_ids, mfT_i, mlpT_i, mfT_u, mlpT_u,
               rag_mf_i, rag_mlp_i, rag_mf_u, rag_mlp_u):
  d = mfT_i.shape[0]
  i32 = jnp.int32

  mesh = plsc.VectorSubcoreMesh(core_axis_name="c", subcore_axis_name="s",
                                num_cores=_NC, num_subcores=_NS)

  @functools.partial(
      pl.kernel,
      out_type=jax.ShapeDtypeStruct((_OUT_ROWS, 2 * d), jnp.float32),
      mesh=mesh,
      compiler_params=pltpu.CompilerParams(use_tc_tiling_on_sc=True,
                                           needs_layout_passes=False),
      scratch_types=[
          pltpu.VMEM((2048,), i32),
          pltpu.VMEM((128, _CAP_I), i32),
          pltpu.VMEM((128, _CAP_U), i32),
          pltpu.VMEM((128,), i32),
          pltpu.VMEM((128,), i32),
          pltpu.VMEM((d, _CW), jnp.float32),
          pltpu.VMEM((d, _CW), jnp.float32),
          pltpu.VMEM((d, _CW), jnp.float32),
          pltpu.VMEM((d, _CW), jnp.float32),
          pltpu.VMEM((_CAP_I, 2 * d), jnp.float32),
          pltpu.VMEM((16, 8), i32),
          pltpu.SemaphoreType.DMA,
          pltpu.SemaphoreType.DMA,
      ],
  )
  def k(item_ids_h, user_ids_h, mfTi_h, mlpTi_h, mfTu_h, mlpTu_h,
        ragA_i_h, ragB_i_h, ragA_u_h, ragB_u_h, out_h,
        idbuf, match_i, match_u, tails_i, tails_u,
        slab_a0, slab_b0, slab_a1, slab_b1, rowbuf, posidx,
        semS, semW):
    wid = lax.axis_index("s") * _NC + lax.axis_index("c")
    nch = 122 + jnp.where(wid >= 30, 1, 0)
    start = 122 * wid + jnp.maximum(wid - 30, 0)
    lo = start * _CW
    hi = lo + nch * _CW + jnp.where(wid == 31, _RAG, 0)
    lanes = lax.broadcasted_iota(i32, (16,), 0)
    zeros16 = jnp.zeros((16,), i32)
    trash16 = jnp.full((16,), _TRASH, i32)

    for k16 in range(8):
      tails_i[pl.ds(16 * k16, 16)] = zeros16
      tails_u[pl.ds(16 * k16, 16)] = zeros16

    def scan_list(ids_h, nblk, pos_base, match, tails, cap):
      def blk(s, carry):
        pltpu.sync_copy(
            ids_h.at[pl.ds(pl.multiple_of(s * 2048, 2048), 2048)], idbuf)
        def step(t, c2):
          idv = idbuf[pl.ds(t * 16, 16)]
          posv = pos_base + s * 2048 + t * 16 + lanes
          m = (idv >= lo) & (idv < hi)
          local = idv - lo
          bkt = lax.shift_right_logical(local, 8)
          rank, lastm = plsc.scan_count(bkt, mask=m)
          tl = plsc.load_gather(tails, [bkt], mask=m)
          wr = tl + rank - 1
          mw = m & (wr < cap)
          packed = posv * 256 + (local & 255)
          plsc.store_scatter(match, [bkt, wr], packed, mask=mw)
          plsc.addupdate_scatter(tails, [bkt], rank, mask=m & lastm)
          return c2
        return lax.fori_loop(0, 128, step, carry)
      lax.fori_loop(0, nblk, blk, 0)

    scan_list(item_ids_h, _BL // 2048, 0, match_i, tails_i, _CAP_I)
    scan_list(user_ids_h, _B // 2048, _BL, match_u, tails_u, _CAP_U)

    def drain(n):
      for s in range(16):
        @pl.when(s < n)
        def _():
          pltpu.make_async_copy(rowbuf.at[pl.ds(0, 8)],
                                out_h.at[pl.ds(0, 8)], semW).wait()

    def process(cnt):
      for k16 in range(8):
        lv = k16 * 16 + lanes
        plsc.store_scatter(posidx, [lax.shift_right_logical(lv, 3), lv & 7],
                           trash16)
      jv_cnt = cnt

      def grp(g, c3, match, j, sa, sb):
        lv = g * 16 + lanes
        mk = lv < jv_cnt
        jb = jnp.full((16,), j, i32)
        packed = plsc.load_gather(match, [jb, lv], mask=mk)
        posv = lax.shift_right_logical(packed, 8)
        localv = packed & 255
        plsc.store_scatter(posidx, [lax.shift_right_logical(lv, 3), lv & 7],
                           posv, mask=mk)
        for r in range(0, d, 1):
          rb = jnp.full((16,), r, i32)
          va = plsc.load_gather(sa, [rb, localv], mask=mk)
          plsc.store_scatter(rowbuf, [lv, rb], va, mask=mk)
          vb = plsc.load_gather(sb, [rb, localv], mask=mk)
          plsc.store_scatter(rowbuf, [lv, rb + d], vb, mask=mk)
        return c3
      return grp

    def gather_pass(tA, tB, ragA, ragB, match, tails, cap):
      def cntf(c):
        cv = plsc.load_gather(tails, [jnp.full((16,), c, i32)])
        return jnp.minimum(jnp.max(cv), cap)

      def live(c):
        return (c < nch) & (cntf(c) > 0)

      def cond_issue(c, sa, sb):
        @pl.when(live(c))
        def _():
          c0 = pl.multiple_of((start + c) * _CW, 128)
          pltpu.async_copy(tA.at[:, pl.ds(c0, _CW)], sa, semS)
          pltpu.async_copy(tB.at[:, pl.ds(c0, _CW)], sb, semS)

      def cond_wait(c, sa, sb):
        @pl.when(live(c))
        def _():
          c0 = pl.multiple_of((start + c) * _CW, 128)
          pltpu.make_async_copy(tA.at[:, pl.ds(c0, _CW)], sa, semS).wait()
          pltpu.make_async_copy(tB.at[:, pl.ds(c0, _CW)], sb, semS).wait()

      def do_chunk(c, sa, sb, nscat_prev):
        cnt = jnp.where(c < nch, cntf(c), 0)
        drain(nscat_prev)
        cond_wait(c, sa, sb)
        grp = process(cnt)
        lax.fori_loop(0, lax.div(cnt + 15, 16),
                      lambda g, cc: grp(g, cc, match, c, sa, sb), 0)
        nscat = lax.div(cnt + 7, 8)
        for s in range(16):
          @pl.when(s < nscat)
          def _():
            pltpu.async_copy(rowbuf.at[pl.ds(8 * s, 8)],
                             out_h.at[posidx.at[s]], semW)
        return nscat

      cond_issue(0, slab_a0, slab_b0)

      def body(j2, nprev):
        ca = 2 * j2
        cb = 2 * j2 + 1
        cond_issue(cb, slab_a1, slab_b1)
        n0 = do_chunk(ca, slab_a0, slab_b0, nprev)
        cond_issue(cb + 1, slab_a0, slab_b0)
        n1 = do_chunk(cb, slab_a1, slab_b1, n0)
        return n1

      last = lax.fori_loop(0, lax.div(nch + 1, 2), body, 0)
      drain(last)

      @pl.when(wid == 31)
      def _():
        cpa = pltpu.async_copy(ragA, slab_a1, semS)
        cpb = pltpu.async_copy(ragB, slab_b1, semS)
        cpa.wait()
        cpb.wait()
        cnt_vec = plsc.load_gather(tails, [jnp.full((16,), 123, i32)])
        cnt = jnp.minimum(jnp.max(cnt_vec), cap)
        grp = process(cnt)
        lax.fori_loop(0, lax.div(cnt + 15, 16),
                      lambda g, c: grp(g, c, match, 123, slab_a1, slab_b1),
                      0)
        nscat = lax.div(cnt + 7, 8)
        for s in range(16):
          @pl.when(s < nscat)
          def _():
            pltpu.async_copy(rowbuf.at[pl.ds(8 * s, 8)],
                             out_h.at[posidx.at[s]], semW)
        drain(nscat)

    gather_pass(mfTi_h, mlpTi_h, ragA_i_h, ragB_i_h, match_i, tails_i,
                _CAP_I)
    gather_pass(mfTu_h, mlpTu_h, ragA_u_h, ragB_u_h, match_u, tails_u,
                _CAP_U)

  return k(item_ids, user_ids, mfT_i, mlpT_i, mfT_u, mlpT_u,
           rag_mf_i, rag_mlp_i, rag_mf_u, rag_mlp_u)


def _tc_body(x_ref, xu_ref, w1u_ref, w1z_ref, b1_ref, w2_ref, b2_ref,
             wmfm_ref, wh2_ref, out_ref, *, bb, ll):
  f32 = jnp.float32
  dot = functools.partial(jnp.dot, preferred_element_type=f32)
  xu = xu_ref[...]
  a = dot(xu, w1u_ref[...]) + b1_ref[...]
  vmf = xu * wmfm_ref[...]
  r_i = lax.broadcasted_iota(jnp.int32, (bb * ll, bb), 0)
  b50 = lax.broadcasted_iota(jnp.int32, (bb * ll, bb), 1) * ll
  p = ((r_i >= b50) & (r_i < b50 + ll)).astype(f32)
  a_exp = dot(p, a)
  vmf_exp = dot(p, vmf)
  x = x_ref[...]
  h1 = jnp.maximum(dot(x, w1z_ref[...]) + a_exp, 0.0)
  h2 = jnp.maximum(dot(h1, w2_ref[...]) + b2_ref[...], 0.0)
  mf_c = jnp.sum(x * vmf_exp, axis=1, keepdims=True)
  out_ref[...] = mf_c + jnp.sum(h2 * wh2_ref[...], axis=1, keepdims=True)


def kernel(user, item, mf_user_em, mf_item_em, mlp_user_em, mlp_item_em,
           W1, b1, W2, b2, Wout):
  b, ll = item.shape
  d = mf_user_em.shape[1]
  bl = b * ll
  f32 = jnp.float32

  item_ids = item.reshape(-1)
  user_ids = user.reshape(-1)

  rag0 = _NCH_FULL * _CW

  def ragpad(t):
    return jnp.pad(t[rag0:, :].T, ((0, 0), (0, _CW - _RAG)))

  out_all = _sc_gather(item_ids, user_ids, mf_item_em.T, mlp_item_em.T,
                       mf_user_em.T, mlp_user_em.T,
                       ragpad(mf_item_em), ragpad(mlp_item_em),
                       ragpad(mf_user_em), ragpad(mlp_user_em))

  zdd = jnp.zeros((d, d), f32)
  w1u = jnp.concatenate([zdd, W1[:d, :]], axis=0)
  w1z = jnp.concatenate([zdd, W1[d:, :]], axis=0)
  b1r = b1.reshape(1, d)
  b2r = b2.reshape(1, d // 2)
  wmfm = jnp.concatenate([Wout[:d, 0],
                          jnp.zeros((d,), f32)]).reshape(1, 2 * d)
  wh2 = Wout[d:, 0].reshape(1, d // 2)

  bb = 128
  n_blocks = b // bb
  rows = bb * ll

  grid_spec = pl.GridSpec(
      grid=(n_blocks,),
      in_specs=[
          pl.BlockSpec((rows, 2 * d), lambda i: (i, 0)),
          pl.BlockSpec((bb, 2 * d), lambda i: (_BL // bb + i, 0)),
          pl.BlockSpec((2 * d, d), lambda i: (0, 0)),
          pl.BlockSpec((2 * d, d), lambda i: (0, 0)),
          pl.BlockSpec((1, d), lambda i: (0, 0)),
          pl.BlockSpec((d, d // 2), lambda i: (0, 0)),
          pl.BlockSpec((1, d // 2), lambda i: (0, 0)),
          pl.BlockSpec((1, 2 * d), lambda i: (0, 0)),
          pl.BlockSpec((1, d // 2), lambda i: (0, 0)),
      ],
      out_specs=pl.BlockSpec((rows, 1), lambda i: (i, 0)),
  )

  out_flat = pl.pallas_call(
      functools.partial(_tc_body, bb=bb, ll=ll),
      grid_spec=grid_spec,
      out_shape=jax.ShapeDtypeStruct((bl, 1), jnp.float32),
      compiler_params=pltpu.CompilerParams(
          dimension_semantics=("parallel",)),
  )(out_all, out_all, w1u, w1z, b1r, W2, b2r, wmfm, wh2)

  return out_flat.reshape(b, ll, 1)
